# Initial kernel scaffold; baseline (speedup 1.0000x reference)
import jax
import jax.numpy as jnp
from jax.experimental import pallas as pl


def kernel(input_embedding, fai_x, fai_x_prime, w_1, b_1, w_2, b_2, Wq, Wk, Wv, Wadd, badd, Wfin, bfin):
    raise NotImplementedError("write your pallas kernel here")


# speedup vs baseline: 11.8086x; 11.8086x over previous
import functools
import math

import jax
import jax.numpy as jnp
import numpy as np
from jax import lax
from jax.experimental import pallas as pl
from jax.experimental.pallas import tpu as pltpu

N = 2048
DM = 768
U = 160
NCLS = 16

_idx_sample = np.asarray(
    jax.random.randint(jax.random.key(42), (N, U), 0, N, dtype=jnp.int32))
_counts = np.zeros((N, N), dtype=np.float32)
np.add.at(_counts, (np.arange(N)[:, None], _idx_sample), 1.0)
_COUNTS = jnp.asarray(_counts)
del _counts


def _kv_base_body(x_ref, wk_ref, wv_ref, wadd_ref, badd_ref,
                  k_ref, v_ref, base_ref, vsum_ref):
    x = x_ref[...]
    dims = (((1,), (1,)), ((), ()))
    k = lax.dot_general(x, wk_ref[...], dims, preferred_element_type=jnp.float32)
    v = lax.dot_general(x, wv_ref[...], dims, preferred_element_type=jnp.float32)
    base = lax.dot_general(x, wadd_ref[...], dims,
                           preferred_element_type=jnp.float32) + badd_ref[...]
    k_ref[...] = k
    v_ref[...] = v
    base_ref[...] = base

    @pl.when(pl.program_id(0) == 0)
    def _():
        vsum_ref[...] = jnp.zeros_like(vsum_ref)

    vsum_ref[...] += jnp.sum(v, axis=0, keepdims=True)


def _m_body(x_ref, wq_ref, k_ref, c_ref, m_ref):
    dims = (((1,), (1,)), ((), ()))
    q = lax.dot_general(x_ref[...], wq_ref[...], dims,
                        preferred_element_type=jnp.float32)
    s = lax.dot_general(q, k_ref[...], dims, preferred_element_type=jnp.float32)
    c = c_ref[...]
    masked = jnp.where(c > 0.0, s, -jnp.inf)
    m = jnp.max(masked, axis=1) - jnp.sum(s * c, axis=1) * (1.0 / N)
    m_ref[...] = m.reshape(1, -1)


def _select_attn_body(m_ref, x_ref, wq_ref, k_ref, v_ref, vsum_ref, base_ref,
                      ctx_ref, mtop_ref, xsel_ref, upd_ref):
    iota = lax.broadcasted_iota(jnp.int32, (1, N), 1)

    def pick(j, m):
        mx = jnp.max(m)
        idx = jnp.min(jnp.where(m == mx, iota, N))
        mtop_ref[j] = idx
        xsel_ref[pl.ds(j, 1), :] = x_ref[pl.ds(idx, 1), :]
        return jnp.where(iota == idx, -jnp.inf, m)

    lax.fori_loop(0, U, pick, m_ref[...], unroll=False)

    dims = (((1,), (1,)), ((), ()))
    q_sel = lax.dot_general(xsel_ref[...], wq_ref[...], dims,
                            preferred_element_type=jnp.float32)
    scores = lax.dot_general(q_sel, k_ref[...], dims,
                             preferred_element_type=jnp.float32)
    scores = scores * (1.0 / math.sqrt(DM))
    scores = scores - jnp.max(scores, axis=1, keepdims=True)
    e = jnp.exp(scores)
    attn = e / jnp.sum(e, axis=1, keepdims=True)
    upd_ref[...] = lax.dot_general(
        attn, v_ref[...], (((1,), (0,)), ((), ())),
        preferred_element_type=jnp.float32)

    vmean = vsum_ref[...] * (1.0 / N)
    ctx_ref[...] = base_ref[...] + vmean

    def scatter(j, _):
        idx = mtop_ref[j]
        ctx_ref[pl.ds(idx, 1), :] = (upd_ref[pl.ds(j, 1), :]
                                     + base_ref[pl.ds(idx, 1), :])
        return 0

    lax.fori_loop(0, U, scatter, 0, unroll=False)


def _matvec_body(ctx_ref, w_ref, bfin_ref, out_ref):
    @pl.when(pl.program_id(0) == 0)
    def _():
        out_ref[...] = bfin_ref[...]

    out_ref[...] += lax.dot_general(
        ctx_ref[...], w_ref[...], (((1,), (1,)), ((), ())),
        preferred_element_type=jnp.float32)


def kernel(input_embedding, fai_x, fai_x_prime, w_1, b_1, w_2, b_2,
           Wq, Wk, Wv, Wadd, badd, Wfin, bfin):
    x = input_embedding.reshape(N, DM)
    badd2 = badd.reshape(1, DM)
    bfin2 = bfin.reshape(1, NCLS)

    blk = 256
    nblk = N // blk
    f32 = jnp.float32

    k, v, base, vsum = pl.pallas_call(
        _kv_base_body,
        grid=(nblk,),
        in_specs=[
            pl.BlockSpec((blk, DM), lambda i: (i, 0)),
            pl.BlockSpec((DM, DM), lambda i: (0, 0)),
            pl.BlockSpec((DM, DM), lambda i: (0, 0)),
            pl.BlockSpec((DM, DM), lambda i: (0, 0)),
            pl.BlockSpec((1, DM), lambda i: (0, 0)),
        ],
        out_specs=[
            pl.BlockSpec((blk, DM), lambda i: (i, 0)),
            pl.BlockSpec((blk, DM), lambda i: (i, 0)),
            pl.BlockSpec((blk, DM), lambda i: (i, 0)),
            pl.BlockSpec((1, DM), lambda i: (0, 0)),
        ],
        out_shape=[
            jax.ShapeDtypeStruct((N, DM), f32),
            jax.ShapeDtypeStruct((N, DM), f32),
            jax.ShapeDtypeStruct((N, DM), f32),
            jax.ShapeDtypeStruct((1, DM), f32),
        ],
    )(x, Wk, Wv, Wadd, badd2)

    m = pl.pallas_call(
        _m_body,
        grid=(nblk,),
        in_specs=[
            pl.BlockSpec((blk, DM), lambda i: (i, 0)),
            pl.BlockSpec((DM, DM), lambda i: (0, 0)),
            pl.BlockSpec((N, DM), lambda i: (0, 0)),
            pl.BlockSpec((blk, N), lambda i: (i, 0)),
        ],
        out_specs=pl.BlockSpec((1, blk), lambda i: (0, i)),
        out_shape=jax.ShapeDtypeStruct((1, N), f32),
    )(x, Wq, k, _COUNTS)

    ctx, _ = pl.pallas_call(
        _select_attn_body,
        grid=(1,),
        in_specs=[
            pl.BlockSpec((1, N), lambda i: (0, 0)),
            pl.BlockSpec((N, DM), lambda i: (0, 0)),
            pl.BlockSpec((DM, DM), lambda i: (0, 0)),
            pl.BlockSpec((N, DM), lambda i: (0, 0)),
            pl.BlockSpec((N, DM), lambda i: (0, 0)),
            pl.BlockSpec((1, DM), lambda i: (0, 0)),
            pl.BlockSpec((N, DM), lambda i: (0, 0)),
        ],
        out_specs=[
            pl.BlockSpec((N, DM), lambda i: (0, 0)),
            pl.BlockSpec(memory_space=pltpu.SMEM),
        ],
        out_shape=[
            jax.ShapeDtypeStruct((N, DM), f32),
            jax.ShapeDtypeStruct((U,), jnp.int32),
        ],
        scratch_shapes=[
            pltpu.VMEM((U, DM), f32),
            pltpu.VMEM((U, DM), f32),
        ],
    )(m, x, Wq, k, v, vsum, base)

    ctx_flat = ctx.reshape(1, N * DM)
    cblk = N * DM // 16
    out = pl.pallas_call(
        _matvec_body,
        grid=(16,),
        in_specs=[
            pl.BlockSpec((1, cblk), lambda i: (0, i)),
            pl.BlockSpec((NCLS, cblk), lambda i: (0, i)),
            pl.BlockSpec((1, NCLS), lambda i: (0, 0)),
        ],
        out_specs=pl.BlockSpec((1, NCLS), lambda i: (0, 0)),
        out_shape=jax.ShapeDtypeStruct((1, NCLS), f32),
    )(ctx_flat, Wfin, bfin2)

    return out


# speedup vs baseline: 16.7977x; 1.4225x over previous
import functools
import math

import jax
import jax.numpy as jnp
import numpy as np
from jax import lax
from jax.experimental import pallas as pl
from jax.experimental.pallas import tpu as pltpu

N = 2048
DM = 768
U = 160
NCLS = 16


def _tf2x32(k1, k2, x1, x2):
    def rotl(x, d):
        return ((x << np.uint32(d)) | (x >> np.uint32(32 - d))).astype(np.uint32)

    def rounds(v0, v1, rots):
        for r in rots:
            v0 = (v0 + v1).astype(np.uint32)
            v1 = v0 ^ rotl(v1, r)
        return v0, v1

    rot0, rot1 = (13, 15, 26, 6), (17, 29, 16, 24)
    ks = [np.uint32(k1), np.uint32(k2)]
    ks.append(np.uint32(ks[0] ^ ks[1] ^ np.uint32(0x1BD11BDA)))
    x0 = (np.asarray(x1, np.uint32) + ks[0]).astype(np.uint32)
    y0 = (np.asarray(x2, np.uint32) + ks[1]).astype(np.uint32)
    for i, rots in enumerate((rot0, rot1, rot0, rot1, rot0)):
        x0, y0 = rounds(x0, y0, rots)
        x0 = (x0 + ks[(i + 1) % 3]).astype(np.uint32)
        y0 = (y0 + ks[(i + 2) % 3] + np.uint32(i + 1)).astype(np.uint32)
    return x0, y0


def _index_sample_np():
    b1, b2 = _tf2x32(0, 42, np.zeros(2, np.uint32), np.arange(2, dtype=np.uint32))
    size = N * U
    c1, c2 = np.zeros(size, np.uint32), np.arange(size, dtype=np.uint32)
    lo1, lo2 = _tf2x32(b1[1], b2[1], c1, c2)
    return ((lo1 ^ lo2) % np.uint32(N)).astype(np.int64).reshape(N, U)


_counts = np.zeros((N, N), dtype=np.int32)
np.add.at(_counts, (np.arange(N)[:, None], _index_sample_np()), 1)
_COUNTS = _counts.astype(np.int8)
del _counts


def _kv_base_body(x_ref, wk_ref, wv_ref, wadd_ref, badd_ref,
                  k_ref, v_ref, base_ref, vsum_ref):
    x = x_ref[...]
    dims = (((1,), (1,)), ((), ()))
    k = lax.dot_general(x, wk_ref[...], dims, preferred_element_type=jnp.float32)
    v = lax.dot_general(x, wv_ref[...], dims, preferred_element_type=jnp.float32)
    base = lax.dot_general(x, wadd_ref[...], dims,
                           preferred_element_type=jnp.float32) + badd_ref[...]
    k_ref[...] = k
    v_ref[...] = v
    base_ref[...] = base

    @pl.when(pl.program_id(0) == 0)
    def _():
        vsum_ref[...] = jnp.zeros_like(vsum_ref)

    vsum_ref[...] += jnp.sum(v, axis=0, keepdims=True)


def _m_body(x_ref, wq_ref, k_ref, c_ref, m_ref):
    dims = (((1,), (1,)), ((), ()))
    q = lax.dot_general(x_ref[...], wq_ref[...], dims,
                        preferred_element_type=jnp.float32)
    s = lax.dot_general(q, k_ref[...], dims, preferred_element_type=jnp.float32)
    c = c_ref[...].astype(jnp.float32)
    masked = jnp.where(c > 0.0, s, -jnp.inf)
    m = jnp.max(masked, axis=1) - jnp.sum(s * c, axis=1) * (1.0 / N)
    m_ref[...] = m.reshape(1, -1)


def _prefix_incl(x):
    s = 1
    while s < N:
        shifted = jnp.concatenate(
            [jnp.zeros((1, s), x.dtype), x[:, :N - s]], axis=1)
        x = x + shifted
        s *= 2
    return x


def _select_attn_body(m_ref, x_ref, wq_ref, k_ref, v_ref, vsum_ref, base_ref,
                      ctx_ref, p_ref):
    m = m_ref[...]
    b = lax.bitcast_convert_type(m, jnp.int32)
    imin = jnp.int32(-2147483648)
    key = jnp.where(b >= 0, b, imin - b)

    cnt_nonneg = jnp.sum((key >= 0).astype(jnp.int32))
    t0 = jnp.where(cnt_nonneg >= U, jnp.int32(0), imin)

    def bit_step(i, t):
        cand = t + lax.shift_left(jnp.int32(1), 30 - i)
        cnt = jnp.sum((key >= cand).astype(jnp.int32))
        return jnp.where(cnt >= U, cand, t)

    t = lax.fori_loop(0, 31, bit_step, t0, unroll=True)

    gt = (key > t)
    eq = (key == t)
    need = U - jnp.sum(gt.astype(jnp.int32))
    eq_i = eq.astype(jnp.int32)
    eq_excl = _prefix_incl(eq_i) - eq_i
    sel = gt | (eq & (eq_excl < need))
    sel_i = sel.astype(jnp.int32)
    pos = _prefix_incl(sel_i) - sel_i

    sel_f = sel.astype(jnp.float32)
    rows = 8
    for r in range(U // rows):
        riota = lax.broadcasted_iota(jnp.int32, (rows, 1), 0) + r * rows
        chunk = (pos == riota).astype(jnp.float32) * sel_f
        p_ref[pl.ds(r * rows, rows), :] = chunk

    p = p_ref[...]
    x_sel = lax.dot_general(p, x_ref[...], (((1,), (0,)), ((), ())),
                            preferred_element_type=jnp.float32)
    dims = (((1,), (1,)), ((), ()))
    q_sel = lax.dot_general(x_sel, wq_ref[...], dims,
                            preferred_element_type=jnp.float32)
    scores = lax.dot_general(q_sel, k_ref[...], dims,
                             preferred_element_type=jnp.float32)
    scores = scores * (1.0 / math.sqrt(DM))
    scores = scores - jnp.max(scores, axis=1, keepdims=True)
    e = jnp.exp(scores)
    attn = e / jnp.sum(e, axis=1, keepdims=True)
    upd = lax.dot_general(attn, v_ref[...], (((1,), (0,)), ((), ())),
                          preferred_element_type=jnp.float32)

    vmean = vsum_ref[...] * (1.0 / N)
    delta = lax.dot_general(p, upd - vmean, (((0,), (0,)), ((), ())),
                            preferred_element_type=jnp.float32)
    ctx_ref[...] = base_ref[...] + vmean + delta


def _matvec_body(ctx_ref, w_ref, bfin_ref, out_ref):
    @pl.when(pl.program_id(0) == 0)
    def _():
        out_ref[...] = bfin_ref[...]

    out_ref[...] += lax.dot_general(
        ctx_ref[...], w_ref[...], (((1,), (1,)), ((), ())),
        preferred_element_type=jnp.float32)


def kernel(input_embedding, fai_x, fai_x_prime, w_1, b_1, w_2, b_2,
           Wq, Wk, Wv, Wadd, badd, Wfin, bfin):
    x = input_embedding.reshape(N, DM)
    badd2 = badd.reshape(1, DM)
    bfin2 = bfin.reshape(1, NCLS)

    blk = 256
    nblk = N // blk
    f32 = jnp.float32

    k, v, base, vsum = pl.pallas_call(
        _kv_base_body,
        grid=(nblk,),
        in_specs=[
            pl.BlockSpec((blk, DM), lambda i: (i, 0)),
            pl.BlockSpec((DM, DM), lambda i: (0, 0)),
            pl.BlockSpec((DM, DM), lambda i: (0, 0)),
            pl.BlockSpec((DM, DM), lambda i: (0, 0)),
            pl.BlockSpec((1, DM), lambda i: (0, 0)),
        ],
        out_specs=[
            pl.BlockSpec((blk, DM), lambda i: (i, 0)),
            pl.BlockSpec((blk, DM), lambda i: (i, 0)),
            pl.BlockSpec((blk, DM), lambda i: (i, 0)),
            pl.BlockSpec((1, DM), lambda i: (0, 0)),
        ],
        out_shape=[
            jax.ShapeDtypeStruct((N, DM), f32),
            jax.ShapeDtypeStruct((N, DM), f32),
            jax.ShapeDtypeStruct((N, DM), f32),
            jax.ShapeDtypeStruct((1, DM), f32),
        ],
    )(x, Wk, Wv, Wadd, badd2)

    m = pl.pallas_call(
        _m_body,
        grid=(nblk,),
        in_specs=[
            pl.BlockSpec((blk, DM), lambda i: (i, 0)),
            pl.BlockSpec((DM, DM), lambda i: (0, 0)),
            pl.BlockSpec((N, DM), lambda i: (0, 0)),
            pl.BlockSpec((blk, N), lambda i: (i, 0)),
        ],
        out_specs=pl.BlockSpec((1, blk), lambda i: (0, i)),
        out_shape=jax.ShapeDtypeStruct((1, N), f32),
    )(x, Wq, k, _COUNTS)

    ctx = pl.pallas_call(
        _select_attn_body,
        grid=(1,),
        in_specs=[
            pl.BlockSpec((1, N), lambda i: (0, 0)),
            pl.BlockSpec((N, DM), lambda i: (0, 0)),
            pl.BlockSpec((DM, DM), lambda i: (0, 0)),
            pl.BlockSpec((N, DM), lambda i: (0, 0)),
            pl.BlockSpec((N, DM), lambda i: (0, 0)),
            pl.BlockSpec((1, DM), lambda i: (0, 0)),
            pl.BlockSpec((N, DM), lambda i: (0, 0)),
        ],
        out_specs=pl.BlockSpec((N, DM), lambda i: (0, 0)),
        out_shape=jax.ShapeDtypeStruct((N, DM), f32),
        scratch_shapes=[
            pltpu.VMEM((U, N), f32),
        ],
    )(m, x, Wq, k, v, vsum, base)

    ctx_flat = ctx.reshape(1, N * DM)
    cblk = N * DM // 16
    out = pl.pallas_call(
        _matvec_body,
        grid=(16,),
        in_specs=[
            pl.BlockSpec((1, cblk), lambda i: (0, i)),
            pl.BlockSpec((NCLS, cblk), lambda i: (0, i)),
            pl.BlockSpec((1, NCLS), lambda i: (0, 0)),
        ],
        out_specs=pl.BlockSpec((1, NCLS), lambda i: (0, 0)),
        out_shape=jax.ShapeDtypeStruct((1, NCLS), f32),
    )(ctx_flat, Wfin, bfin2)

    return out


# speedup vs baseline: 22.5679x; 1.3435x over previous
import math

import jax
import jax.numpy as jnp
import numpy as np
from jax import lax
from jax.experimental import pallas as pl
from jax.experimental.pallas import tpu as pltpu

N = 2048
DM = 768
U = 160
NCLS = 16


def _tf2x32(k1, k2, x1, x2):
    def rotl(x, d):
        return ((x << np.uint32(d)) | (x >> np.uint32(32 - d))).astype(np.uint32)

    def rounds(v0, v1, rots):
        for r in rots:
            v0 = (v0 + v1).astype(np.uint32)
            v1 = v0 ^ rotl(v1, r)
        return v0, v1

    rot0, rot1 = (13, 15, 26, 6), (17, 29, 16, 24)
    ks = [np.uint32(k1), np.uint32(k2)]
    ks.append(np.uint32(ks[0] ^ ks[1] ^ np.uint32(0x1BD11BDA)))
    x0 = (np.asarray(x1, np.uint32) + ks[0]).astype(np.uint32)
    y0 = (np.asarray(x2, np.uint32) + ks[1]).astype(np.uint32)
    for i, rots in enumerate((rot0, rot1, rot0, rot1, rot0)):
        x0, y0 = rounds(x0, y0, rots)
        x0 = (x0 + ks[(i + 1) % 3]).astype(np.uint32)
        y0 = (y0 + ks[(i + 2) % 3] + np.uint32(i + 1)).astype(np.uint32)
    return x0, y0


def _index_sample_np():
    b1, b2 = _tf2x32(0, 42, np.zeros(2, np.uint32), np.arange(2, dtype=np.uint32))
    size = N * U
    c1, c2 = np.zeros(size, np.uint32), np.arange(size, dtype=np.uint32)
    lo1, lo2 = _tf2x32(b1[1], b2[1], c1, c2)
    return ((lo1 ^ lo2) % np.uint32(N)).astype(np.int64).reshape(N, U)


_counts = np.zeros((N, N), dtype=np.int32)
np.add.at(_counts, (np.arange(N)[:, None], _index_sample_np()), 1)
_COUNTS = _counts.astype(np.int8)
del _counts

_BLK = 256
_NBLK = N // _BLK


def _kv_body(x_ref, wk_ref, wv_ref, k_ref, v_ref, vsum_ref):
    x = x_ref[...]
    dims = (((1,), (1,)), ((), ()))
    k = lax.dot_general(x, wk_ref[...], dims, preferred_element_type=jnp.float32)
    v = lax.dot_general(x, wv_ref[...], dims, preferred_element_type=jnp.float32)
    k_ref[...] = k
    v_ref[...] = v

    @pl.when(pl.program_id(0) == 0)
    def _():
        vsum_ref[...] = jnp.zeros_like(vsum_ref)

    vsum_ref[...] += jnp.sum(v, axis=0, keepdims=True)


def _prefix_incl(x):
    s = 1
    while s < N:
        shifted = jnp.concatenate(
            [jnp.zeros((1, s), x.dtype), x[:, :N - s]], axis=1)
        x = x + shifted
        s *= 2
    return x


def _select_attn_body(x_ref, wq_ref, k_ref, v_ref, vsum_ref, c_ref,
                      p_ref, updm_ref, m_ref):
    dims = (((1,), (1,)), ((), ()))
    kf = k_ref[...]

    for blk in range(_NBLK):
        xb = x_ref[pl.ds(blk * _BLK, _BLK), :]
        q = lax.dot_general(xb, wq_ref[...], dims,
                            preferred_element_type=jnp.float32)
        s = lax.dot_general(q, kf, dims, preferred_element_type=jnp.float32)
        c = c_ref[pl.ds(blk * _BLK, _BLK), :].astype(jnp.float32)
        masked = jnp.where(c > 0.0, s, -jnp.inf)
        mb = jnp.max(masked, axis=1) - jnp.sum(s * c, axis=1) * (1.0 / N)
        m_ref[pl.ds(0, 1), pl.ds(blk * _BLK, _BLK)] = mb.reshape(1, _BLK)

    m = m_ref[...]
    b = lax.bitcast_convert_type(m, jnp.int32)
    imin = jnp.int32(-2147483648)
    key = jnp.where(b >= 0, b, imin - b)

    cnt_nonneg = jnp.sum((key >= 0).astype(jnp.int32))
    t0 = jnp.where(cnt_nonneg >= U, jnp.int32(0), imin)

    def bit_step(i, t):
        cand = t + lax.shift_left(jnp.int32(1), 30 - i)
        cnt = jnp.sum((key >= cand).astype(jnp.int32))
        return jnp.where(cnt >= U, cand, t)

    t = lax.fori_loop(0, 31, bit_step, t0, unroll=True)

    gt = (key > t)
    eq = (key == t)
    need = U - jnp.sum(gt.astype(jnp.int32))
    eq_i = eq.astype(jnp.int32)
    eq_excl = _prefix_incl(eq_i) - eq_i
    sel = gt | (eq & (eq_excl < need))
    sel_i = sel.astype(jnp.int32)
    pos = _prefix_incl(sel_i) - sel_i

    sel_f = sel.astype(jnp.float32)
    rows = 8
    for r in range(U // rows):
        riota = lax.broadcasted_iota(jnp.int32, (rows, 1), 0) + r * rows
        chunk = (pos == riota).astype(jnp.float32) * sel_f
        p_ref[pl.ds(r * rows, rows), :] = chunk

    p = p_ref[...]
    x_sel = lax.dot_general(p, x_ref[...], (((1,), (0,)), ((), ())),
                            preferred_element_type=jnp.float32)
    q_sel = lax.dot_general(x_sel, wq_ref[...], dims,
                            preferred_element_type=jnp.float32)
    scores = lax.dot_general(q_sel, kf, dims,
                             preferred_element_type=jnp.float32)
    scores = scores * (1.0 / math.sqrt(DM))
    scores = scores - jnp.max(scores, axis=1, keepdims=True)
    e = jnp.exp(scores)
    attn = e / jnp.sum(e, axis=1, keepdims=True)
    upd = lax.dot_general(attn, v_ref[...], (((1,), (0,)), ((), ())),
                          preferred_element_type=jnp.float32)
    updm_ref[...] = upd - vsum_ref[...] * (1.0 / N)


def _matvec_body(x_ref, wadd_ref, badd_ref, vsum_ref, p_ref, updm_ref,
                 w_ref, bfin_ref, out_ref):
    base = lax.dot_general(x_ref[...], wadd_ref[...], (((1,), (1,)), ((), ())),
                           preferred_element_type=jnp.float32) + badd_ref[...]
    delta = lax.dot_general(p_ref[...], updm_ref[...], (((0,), (0,)), ((), ())),
                            preferred_element_type=jnp.float32)
    ctx = base + vsum_ref[...] * (1.0 / N) + delta
    flat = jnp.reshape(ctx, (1, ctx.shape[0] * DM))

    @pl.when(pl.program_id(0) == 0)
    def _():
        out_ref[...] = bfin_ref[...]

    out_ref[...] += lax.dot_general(
        flat, w_ref[...], (((1,), (1,)), ((), ())),
        preferred_element_type=jnp.float32)


def kernel(input_embedding, fai_x, fai_x_prime, w_1, b_1, w_2, b_2,
           Wq, Wk, Wv, Wadd, badd, Wfin, bfin):
    x = input_embedding.reshape(N, DM)
    badd2 = badd.reshape(1, DM)
    bfin2 = bfin.reshape(1, NCLS)
    f32 = jnp.float32

    k, v, vsum = pl.pallas_call(
        _kv_body,
        grid=(_NBLK,),
        in_specs=[
            pl.BlockSpec((_BLK, DM), lambda i: (i, 0)),
            pl.BlockSpec((DM, DM), lambda i: (0, 0)),
            pl.BlockSpec((DM, DM), lambda i: (0, 0)),
        ],
        out_specs=[
            pl.BlockSpec((_BLK, DM), lambda i: (i, 0)),
            pl.BlockSpec((_BLK, DM), lambda i: (i, 0)),
            pl.BlockSpec((1, DM), lambda i: (0, 0)),
        ],
        out_shape=[
            jax.ShapeDtypeStruct((N, DM), f32),
            jax.ShapeDtypeStruct((N, DM), f32),
            jax.ShapeDtypeStruct((1, DM), f32),
        ],
    )(x, Wk, Wv)

    p, updm = pl.pallas_call(
        _select_attn_body,
        grid=(1,),
        in_specs=[
            pl.BlockSpec((N, DM), lambda i: (0, 0)),
            pl.BlockSpec((DM, DM), lambda i: (0, 0)),
            pl.BlockSpec((N, DM), lambda i: (0, 0)),
            pl.BlockSpec((N, DM), lambda i: (0, 0)),
            pl.BlockSpec((1, DM), lambda i: (0, 0)),
            pl.BlockSpec((N, N), lambda i: (0, 0)),
        ],
        out_specs=[
            pl.BlockSpec((U, N), lambda i: (0, 0)),
            pl.BlockSpec((U, DM), lambda i: (0, 0)),
        ],
        out_shape=[
            jax.ShapeDtypeStruct((U, N), f32),
            jax.ShapeDtypeStruct((U, DM), f32),
        ],
        scratch_shapes=[
            pltpu.VMEM((1, N), f32),
        ],
    )(x, Wq, k, v, vsum, jnp.asarray(_COUNTS))

    nout = 16
    cblk = N // nout
    out = pl.pallas_call(
        _matvec_body,
        grid=(nout,),
        in_specs=[
            pl.BlockSpec((cblk, DM), lambda i: (i, 0)),
            pl.BlockSpec((DM, DM), lambda i: (0, 0)),
            pl.BlockSpec((1, DM), lambda i: (0, 0)),
            pl.BlockSpec((1, DM), lambda i: (0, 0)),
            pl.BlockSpec((U, cblk), lambda i: (0, i)),
            pl.BlockSpec((U, DM), lambda i: (0, 0)),
            pl.BlockSpec((NCLS, cblk * DM), lambda i: (0, i)),
            pl.BlockSpec((1, NCLS), lambda i: (0, 0)),
        ],
        out_specs=pl.BlockSpec((1, NCLS), lambda i: (0, 0)),
        out_shape=jax.ShapeDtypeStruct((1, NCLS), f32),
    )(x, Wadd, badd2, vsum, p, updm, Wfin, bfin2)

    return out
